# Initial kernel scaffold; baseline (speedup 1.0000x reference)
import jax
import jax.numpy as jnp
from jax.experimental import pallas as pl


def kernel(x, edge_weight, edge_index, batch, W1, att_src1, att_dst1, bias1, ht1_W, ht1_b, pool1_w, W2, att_src2, att_dst2, bias2, ht2_W, ht2_b, pool2_w, W3, att_src3, att_dst3, bias3, ht3_W, ht3_b, lin1_W, lin1_b, lin2_W, lin2_b):
    raise NotImplementedError("write your pallas kernel here")


# speedup vs baseline: 1.0102x; 1.0102x over previous
import functools

import jax
import jax.numpy as jnp
from jax import lax
from jax.experimental import pallas as pl
from jax.experimental.pallas import tpu as pltpu

H = 3
C = 256


def _mm_body(x_ref, w_ref, o_ref):
    o_ref[...] = jnp.dot(x_ref[...], w_ref[...],
                         preferred_element_type=jnp.float32)


def pallas_matmul(x, w, bm=1000):
    M, K = x.shape
    K2, N = w.shape
    grid = (M // bm,)
    return pl.pallas_call(
        _mm_body,
        grid=grid,
        in_specs=[
            pl.BlockSpec((bm, K), lambda i: (i, 0)),
            pl.BlockSpec((K, N), lambda i: (0, 0)),
        ],
        out_specs=pl.BlockSpec((bm, N), lambda i: (i, 0)),
        out_shape=jax.ShapeDtypeStruct((M, N), jnp.float32),
    )(x, w)


def _gat_conv(x, edge_index, W, att_src, att_dst, bias, edge_mask):
    N = x.shape[0]
    loop = jnp.arange(N, dtype=edge_index.dtype)
    row = jnp.concatenate([edge_index[0], loop])
    col = jnp.concatenate([edge_index[1], loop])
    full_mask = jnp.concatenate([edge_mask, jnp.ones((N,), bool)])
    col_m = jnp.where(full_mask, col, N)
    h = pallas_matmul(x, W).reshape(N, H, C)
    a_s = (h * att_src[None, :, :]).sum(-1)
    a_d = (h * att_dst[None, :, :]).sum(-1)
    e = jax.nn.leaky_relu(a_s[row] + a_d[col], 0.2)
    m = jax.ops.segment_max(e, col_m, num_segments=N + 1)
    m = jnp.where(jnp.isfinite(m), m, 0.0)
    ex = jnp.exp(e - m[col_m])
    s = jax.ops.segment_sum(ex, col_m, num_segments=N + 1)
    alpha = ex / (s[col_m] + 1e-16)
    out = jax.ops.segment_sum(h[row] * alpha[:, :, None], col_m,
                              num_segments=N + 1)
    return out[:N].reshape(N, H * C) + bias


def _topk_structure(score, batch, edge_index, ratio, num_graphs, node_mask,
                    edge_mask):
    N = score.shape[0]
    counts = jnp.bincount(jnp.where(node_mask, batch, num_graphs),
                          length=num_graphs + 1)[:num_graphs]
    k = jnp.ceil(ratio * counts.astype(jnp.float32)).astype(counts.dtype)
    key = jnp.where(node_mask, batch.astype(jnp.float32) * 4.0 - score,
                    jnp.inf)
    order = jnp.argsort(key)
    offsets = jnp.concatenate([jnp.zeros((1,), counts.dtype),
                               jnp.cumsum(counts)[:-1]])
    sb = batch[order]
    rank = jnp.arange(N, dtype=counts.dtype) - offsets[sb]
    keep = (rank < k[sb]) & node_mask[order]
    new_mask = jnp.zeros((N,), bool).at[order].set(keep)
    r, c = edge_index[0], edge_index[1]
    new_emask = edge_mask & new_mask[r] & new_mask[c]
    return new_mask, new_emask


def _topk_pool(x, w, batch, edge_index, ratio, num_graphs, node_mask,
               edge_mask):
    score = jnp.tanh((x @ w) / (jnp.linalg.norm(w) + 1e-16))
    new_mask, new_emask = _topk_structure(score, batch, edge_index, ratio,
                                          num_graphs, node_mask, edge_mask)
    return (jnp.where(new_mask[:, None], x * score[:, None], 0.0), new_mask,
            new_emask)


def _global_pools(x, batch, num_graphs, node_mask):
    b = jnp.where(node_mask, batch, num_graphs)
    mx = jax.ops.segment_max(x, b, num_segments=num_graphs + 1)[:num_graphs]
    sm = jax.ops.segment_sum(x, b, num_segments=num_graphs + 1)[:num_graphs]
    cnt = jax.ops.segment_sum(jnp.ones((x.shape[0], 1), x.dtype), b,
                              num_segments=num_graphs + 1)[:num_graphs]
    return jnp.concatenate([mx, sm / jnp.maximum(cnt, 1.0)], axis=1)


def kernel(x, edge_weight, edge_index, batch, W1, att_src1, att_dst1, bias1,
           ht1_W, ht1_b, pool1_w, W2, att_src2, att_dst2, bias2, ht2_W, ht2_b,
           pool2_w, W3, att_src3, att_dst3, bias3, ht3_W, ht3_b, lin1_W,
           lin1_b, lin2_W, lin2_b):
    G = 128
    N = x.shape[0]
    node_mask = jnp.ones((N,), bool)
    edge_mask = jnp.ones((edge_index.shape[1],), bool)
    h = _gat_conv(x, edge_index, W1, att_src1, att_dst1, bias1, edge_mask)
    h = pallas_matmul(h, ht1_W) + ht1_b
    h, node_mask, edge_mask = _topk_pool(h, pool1_w, batch, edge_index, 0.8,
                                         G, node_mask, edge_mask)
    x1 = _global_pools(h, batch, G, node_mask)
    h = _gat_conv(h, edge_index, W2, att_src2, att_dst2, bias2, edge_mask)
    h = pallas_matmul(h, ht2_W) + ht2_b
    h, node_mask, edge_mask = _topk_pool(h, pool2_w, batch, edge_index, 0.5,
                                         G, node_mask, edge_mask)
    x2 = _global_pools(h, batch, G, node_mask)
    h = _gat_conv(h, edge_index, W3, att_src3, att_dst3, bias3, edge_mask)
    h = pallas_matmul(h, ht3_W) + ht3_b
    h, node_mask, edge_mask = _topk_pool(h, pool2_w, batch, edge_index, 0.5,
                                         G, node_mask, edge_mask)
    x3 = _global_pools(h, batch, G, node_mask)
    z = x1 + x2 + x3
    z = jax.nn.relu(z @ lin1_W + lin1_b)
    z = z @ lin2_W + lin2_b
    return jax.nn.sigmoid(z)
